# baseline (speedup 1.0000x reference)
import functools

import jax
import jax.numpy as jnp
from jax import lax
from jax.experimental import pallas as pl
from jax.experimental.pallas import tpu as pltpu
from jax.experimental.pallas import tpu_sc as plsc

N = 10000
H = 64
WAUG = 80
NC, NS = 2, 16
NW = NC * NS
IDX = 128

f32 = jnp.float32
i32 = jnp.int32
U = 2


@functools.lru_cache(maxsize=None)
def _make_seg_sum(n_rows, W, K, n_acc, CK):
    NCK = K // CK
    rz = n_acc // NS
    assert rz * NS == n_acc and rz % 8 == 0 and NCK * CK == K
    mesh = plsc.VectorSubcoreMesh(core_axis_name="c", subcore_axis_name="s",
                                  num_cores=NC, num_subcores=NS)

    @functools.partial(
        pl.kernel,
        mesh=mesh,
        compiler_params=pltpu.CompilerParams(use_tc_tiling_on_sc=False),
        out_type=jax.ShapeDtypeStruct((NC, n_acc, W), f32),
        scratch_types=[
            pltpu.VMEM((CK, IDX), i32),
            pltpu.VMEM((CK, IDX), i32),
            pltpu.VMEM((2, IDX, W), f32),
            pltpu.VMEM_SHARED((n_acc, W), f32),
            pltpu.SemaphoreType.DMA((2,)),
        ],
    )
    def seg_sum(table, src3d, dst3d, zeros, out, src_v, dst_v, rows_v, acc,
                sems):
        c = lax.axis_index("c")
        s = lax.axis_index("s")
        wid = s * NC + c
        pltpu.sync_copy(zeros.at[pl.ds(s * rz, rz)], acc.at[pl.ds(s * rz, rz)])
        plsc.subcore_barrier()

        def outer(tc, carry):
            pltpu.sync_copy(src3d.at[wid].at[pl.ds(tc * CK, CK)], src_v)
            pltpu.sync_copy(dst3d.at[wid].at[pl.ds(tc * CK, CK)], dst_v)
            pltpu.sync_copy(table.at[src_v.at[0]], rows_v.at[0])

            def body(j, carry2):
                b = lax.rem(j, 2)
                d = pltpu.async_copy(table.at[src_v.at[j + 1]],
                                     rows_v.at[1 - b], sems.at[1 - b])
                pltpu.sync_copy(rows_v.at[b], acc.at[dst_v.at[j]], add=True)
                d.wait()
                return carry2

            lax.fori_loop(0, CK - 1, body, 0)
            pltpu.sync_copy(rows_v.at[(CK - 1) % 2], acc.at[dst_v.at[CK - 1]],
                            add=True)
            return carry

        lax.fori_loop(0, NCK, outer, 0)
        plsc.subcore_barrier()
        pltpu.sync_copy(acc.at[pl.ds(s * rz, rz)], out.at[c].at[pl.ds(s * rz, rz)])

    return seg_sum


def _seg_sum(table, src, dst, n_out):
    n_rows, W = table.shape
    a = src.shape[0]
    CK = 16 if a > NW * IDX * 16 else -(-a // (NW * IDX))
    gran = NW * IDX * CK
    a_pad = ((a + gran - 1) // gran) * gran
    n_acc = ((n_out + 1 + 127) // 128) * 128
    if a_pad != a:
        pad = a_pad - a
        src = jnp.concatenate([src, jnp.zeros((pad,), i32)])
        spare = jnp.arange(pad, dtype=i32) % (n_acc - n_out) + n_out
        dst = jnp.concatenate([dst, spare])
    K = a_pad // (NW * IDX)
    fn = _make_seg_sum(n_rows, W, K, n_acc, CK)
    zeros = jnp.zeros((n_acc, W), f32)
    return fn(table, src.reshape(NW, K, IDX), dst.reshape(NW, K, IDX), zeros)


RB = 1000


def _elu(v):
    return jnp.where(v > 0, v, jnp.exp(jnp.minimum(v, 0.0)) - 1.0)


def _dot(a, b):
    return jnp.dot(a, b, preferred_element_type=f32)


def _entry_body(x_ref, wn_ref, wr_ref, b_ref, y_ref, r_ref):
    h = x_ref[...]
    y_ref[...] = _dot(h, wn_ref[...])
    r_ref[...] = _dot(h, wr_ref[...]) + b_ref[...]


def _mid_body(p_ref, r_ref, wn_ref, wr_ref, b_ref, y_ref, r2_ref):
    p = p_ref[...]
    h = _elu(p[0] + p[1] + r_ref[...])
    y_ref[...] = _dot(h, wn_ref[...])
    r2_ref[...] = _dot(h, wr_ref[...]) + b_ref[...]


def _aug_body(p_ref, r_ref, o_ref):
    p = p_ref[...]
    h = _elu(p[0] + p[1] + r_ref[...])
    o_ref[...] = jnp.concatenate(
        [h, jnp.ones((h.shape[0], 1), f32), jnp.zeros((h.shape[0], WAUG - H - 1), f32)],
        axis=1,
    )


def _lvl_entry_body(p_ref, iso_ref, wna_ref, wnb_ref, wra_ref, wrb_ref, b_ref,
                    y_ref, r_ref):
    p = p_ref[...]
    s = p[0] + p[1]
    m = s[:, :H] / jnp.maximum(s[:, H:H + 1], 1.0)
    iso = iso_ref[...]
    y_ref[...] = _dot(m, wna_ref[...]) + _dot(iso, wnb_ref[...])
    r_ref[...] = _dot(m, wra_ref[...]) + _dot(iso, wrb_ref[...]) + b_ref[...]


def _rows_spec(w):
    return pl.BlockSpec((2, RB, w), lambda i: (0, i, 0))


def _full_spec(shape):
    nd = len(shape)
    return pl.BlockSpec(shape, lambda i: (0,) * nd)


def _entry(x, wn, wr, b):
    return pl.pallas_call(
        _entry_body,
        grid=(N // RB,),
        in_specs=[
            pl.BlockSpec((RB, x.shape[1]), lambda i: (i, 0)),
            _full_spec(wn.shape), _full_spec(wr.shape), _full_spec(b.shape),
        ],
        out_specs=[pl.BlockSpec((RB, H), lambda i: (i, 0))] * 2,
        out_shape=[jax.ShapeDtypeStruct((N, H), f32)] * 2,
    )(x, wn, wr, b)


def _mid(p, r, wn, wr, b):
    return pl.pallas_call(
        _mid_body,
        grid=(N // RB,),
        in_specs=[
            _rows_spec(H),
            pl.BlockSpec((RB, H), lambda i: (i, 0)),
            _full_spec(wn.shape), _full_spec(wr.shape), _full_spec(b.shape),
        ],
        out_specs=[pl.BlockSpec((RB, H), lambda i: (i, 0))] * 2,
        out_shape=[jax.ShapeDtypeStruct((N, H), f32)] * 2,
    )(p, r, wn, wr, b)


def _aug(p, r):
    return pl.pallas_call(
        _aug_body,
        grid=(N // RB,),
        in_specs=[_rows_spec(H), pl.BlockSpec((RB, H), lambda i: (i, 0))],
        out_specs=pl.BlockSpec((RB, WAUG), lambda i: (i, 0)),
        out_shape=jax.ShapeDtypeStruct((N, WAUG), f32),
    )(p, r)


def _lvl_entry(p, iso, wna, wnb, wra, wrb, b):
    ni = iso.shape[1]
    return pl.pallas_call(
        _lvl_entry_body,
        grid=(N // RB,),
        in_specs=[
            _rows_spec(WAUG),
            pl.BlockSpec((RB, ni), lambda i: (i, 0)),
            _full_spec(wna.shape), _full_spec(wnb.shape),
            _full_spec(wra.shape), _full_spec(wrb.shape), _full_spec(b.shape),
        ],
        out_specs=[pl.BlockSpec((RB, H), lambda i: (i, 0))] * 2,
        out_shape=[jax.ShapeDtypeStruct((N, H), f32)] * 2,
    )(p, iso, wna, wnb, wra, wrb, b)


def _head_body(x1_ref, x2_ref, x3_ref, f1a_ref, f1b_ref, f1c_ref, b1_ref,
               w2_ref, b2_ref, w3_ref, b3_ref, o_ref):
    a = x1_ref[...]
    x1 = (a[0] + a[1])[:, :H]
    a = x2_ref[...]
    s = a[0] + a[1]
    x2 = s[:, :H] / jnp.maximum(s[:, H:H + 1], 1.0)
    a = x3_ref[...]
    s = a[0] + a[1]
    x3 = s[:, :H] / jnp.maximum(s[:, H:H + 1], 1.0)
    z = _elu(_dot(x1, f1a_ref[...]) + _dot(x2, f1b_ref[...])
             + _dot(x3, f1c_ref[...]) + b1_ref[...])
    z = _elu(_dot(z, w2_ref[...]) + b2_ref[...])
    z = _dot(z, w3_ref[...]) + b3_ref[...]
    m = jnp.max(z, axis=1, keepdims=True)
    lse = jnp.log(jnp.sum(jnp.exp(z - m), axis=1, keepdims=True)) + m
    o_ref[...] = z - lse


def _head(x1s, x2s, x3s, f1a, f1b, f1c, b1, w2, b2, w3, b3):
    args = (x1s, x2s, x3s, f1a, f1b, f1c, b1, w2, b2, w3, b3)

    def spec(shape):
        return pl.BlockSpec(shape, functools.partial(lambda n: (0,) * n, len(shape)))

    return pl.pallas_call(
        _head_body,
        in_specs=[spec(a.shape) for a in args],
        out_specs=pl.BlockSpec((64, 10), lambda: (0, 0)),
        out_shape=jax.ShapeDtypeStruct((64, 10), f32),
    )(*args)


def kernel(x, edge_index, batch, assignment_index_2, iso_type_2, edge_index_2,
           batch_2, assignment_index_3, iso_type_3, edge_index_3, batch_3,
           W1_root, W1_rel, b1, W2_root, W2_rel, b2, W3_root, W3_rel, b3,
           W4_root, W4_rel, b4, W5_root, W5_rel, b5, W6_root, W6_rel, b6,
           W7_root, W7_rel, b7, fc1_W, fc1_b, fc2_W, fc2_b, fc3_W, fc3_b):
    iota = jnp.arange(N, dtype=i32)
    src1, dst1 = edge_index[0], edge_index[1]

    y, r = _entry(x, W1_rel, W1_root, b1.reshape(1, H))
    p = _seg_sum(y, src1, dst1, N)
    y, r = _mid(p, r, W2_rel, W2_root, b2.reshape(1, H))
    p = _seg_sum(y, src1, dst1, N)
    y, r = _mid(p, r, W3_rel, W3_root, b3.reshape(1, H))
    p = _seg_sum(y, src1, dst1, N)
    h_aug = _aug(p, r)

    src_all = jnp.concatenate([iota, assignment_index_2[0], assignment_index_3[0]])
    dst_all = jnp.concatenate([batch, assignment_index_2[1] + 64,
                               assignment_index_3[1] + 10064])
    mega = _seg_sum(h_aug, src_all, dst_all, 20064)
    x1s = mega[:, :64]
    pool2 = mega[:, 64:10064]
    pool3 = mega[:, 10064:20064]

    def level(pool, iso, wroot_e, wrel_e, b_e, wroot_m, wrel_m, b_m, ei, bat):
        y, r = _lvl_entry(pool, iso, wrel_e[:H], wrel_e[H:],
                          wroot_e[:H], wroot_e[H:], b_e.reshape(1, H))
        p = _seg_sum(y, ei[0], ei[1], N)
        y, r = _mid(p, r, wrel_m, wroot_m, b_m.reshape(1, H))
        p = _seg_sum(y, ei[0], ei[1], N)
        haug = _aug(p, r)
        return _seg_sum(haug, iota, bat, 64)[:, :64], p

    x2s, _ = level(pool2, iso_type_2, W4_root, W4_rel, b4, W5_root, W5_rel,
                   b5, edge_index_2, batch_2)
    x3s, _ = level(pool3, iso_type_3, W6_root, W6_rel, b6, W7_root, W7_rel, b7,
                   edge_index_3, batch_3)

    return _head(x1s, x2s, x3s,
                 fc1_W[:H], fc1_W[H:2 * H], fc1_W[2 * H:],
                 fc1_b.reshape(1, H), fc2_W, fc2_b.reshape(1, 32),
                 fc3_W, fc3_b.reshape(1, 10))


# speedup vs baseline: 1.3668x; 1.3668x over previous
import functools

import jax
import jax.numpy as jnp
from jax import lax
from jax.experimental import pallas as pl
from jax.experimental.pallas import tpu as pltpu
from jax.experimental.pallas import tpu_sc as plsc

N = 10000
H = 64
WAUG = 80
NC, NS = 2, 16
NW = NC * NS
IDX = 128

f32 = jnp.float32
i32 = jnp.int32
U = 2


@functools.lru_cache(maxsize=None)
def _make_seg_sum(n_rows, W, K, n_acc, CK):
    NCK = K // CK
    rz = n_acc // NS
    assert rz * NS == n_acc and rz % 8 == 0 and NCK * CK == K
    mesh = plsc.VectorSubcoreMesh(core_axis_name="c", subcore_axis_name="s",
                                  num_cores=NC, num_subcores=NS)

    @functools.partial(
        pl.kernel,
        mesh=mesh,
        compiler_params=pltpu.CompilerParams(use_tc_tiling_on_sc=False),
        out_type=jax.ShapeDtypeStruct((NC, n_acc, W), f32),
        scratch_types=[
            pltpu.VMEM((CK, IDX), i32),
            pltpu.VMEM((CK, IDX), i32),
            pltpu.VMEM((2, IDX, W), f32),
            pltpu.VMEM_SHARED((n_acc, W), f32),
            pltpu.SemaphoreType.DMA((2,)),
        ],
    )
    def seg_sum(table, src3d, dst3d, zeros, out, src_v, dst_v, rows_v, acc,
                sems):
        c = lax.axis_index("c")
        s = lax.axis_index("s")
        wid = s * NC + c
        pltpu.sync_copy(zeros.at[pl.ds(s * rz, rz)], acc.at[pl.ds(s * rz, rz)])
        plsc.subcore_barrier()

        def outer(tc, carry):
            pltpu.sync_copy(src3d.at[wid].at[pl.ds(tc * CK, CK)], src_v)
            pltpu.sync_copy(dst3d.at[wid].at[pl.ds(tc * CK, CK)], dst_v)
            pltpu.sync_copy(table.at[src_v.at[0]], rows_v.at[0])

            def body(j, carry2):
                b = lax.rem(j, 2)
                d = pltpu.async_copy(table.at[src_v.at[j + 1]],
                                     rows_v.at[1 - b], sems.at[1 - b])
                pltpu.sync_copy(rows_v.at[b], acc.at[dst_v.at[j]], add=True)
                d.wait()
                return carry2

            lax.fori_loop(0, CK - 1, body, 0)
            pltpu.sync_copy(rows_v.at[(CK - 1) % 2], acc.at[dst_v.at[CK - 1]],
                            add=True)
            return carry

        lax.fori_loop(0, NCK, outer, 0)
        plsc.subcore_barrier()
        pltpu.sync_copy(acc.at[pl.ds(s * rz, rz)], out.at[c].at[pl.ds(s * rz, rz)])

    return seg_sum


def _seg_sum(table, src, dst, n_out):
    n_rows, W = table.shape
    a = src.shape[0]
    gran = NW * IDX
    a_pad = ((a + gran - 1) // gran) * gran
    CK = a_pad // gran
    n_acc = ((n_out + 1 + 127) // 128) * 128
    if a_pad != a:
        pad = a_pad - a
        src = jnp.concatenate([src, jnp.zeros((pad,), i32)])
        spare = jnp.arange(pad, dtype=i32) % (n_acc - n_out) + n_out
        dst = jnp.concatenate([dst, spare])
    K = a_pad // (NW * IDX)
    fn = _make_seg_sum(n_rows, W, K, n_acc, CK)
    zeros = jnp.zeros((n_acc, W), f32)
    return fn(table, src.reshape(NW, K, IDX), dst.reshape(NW, K, IDX), zeros)


RB = 1000


def _elu(v):
    return jnp.where(v > 0, v, jnp.exp(jnp.minimum(v, 0.0)) - 1.0)


def _dot(a, b):
    return jnp.dot(a, b, preferred_element_type=f32)


def _entry_body(x_ref, wn_ref, wr_ref, b_ref, y_ref, r_ref):
    h = x_ref[...]
    y_ref[...] = _dot(h, wn_ref[...])
    r_ref[...] = _dot(h, wr_ref[...]) + b_ref[...]


def _mid_body(p_ref, r_ref, wn_ref, wr_ref, b_ref, y_ref, r2_ref):
    p = p_ref[...]
    h = _elu(p[0] + p[1] + r_ref[...])
    y_ref[...] = _dot(h, wn_ref[...])
    r2_ref[...] = _dot(h, wr_ref[...]) + b_ref[...]


def _aug_body(p_ref, r_ref, o_ref):
    p = p_ref[...]
    h = _elu(p[0] + p[1] + r_ref[...])
    o_ref[...] = jnp.concatenate(
        [h, jnp.ones((h.shape[0], 1), f32), jnp.zeros((h.shape[0], WAUG - H - 1), f32)],
        axis=1,
    )


def _lvl_entry_body(p_ref, iso_ref, wna_ref, wnb_ref, wra_ref, wrb_ref, b_ref,
                    y_ref, r_ref):
    p = p_ref[...]
    s = p[0] + p[1]
    m = s[:, :H] / jnp.maximum(s[:, H:H + 1], 1.0)
    iso = iso_ref[...]
    y_ref[...] = _dot(m, wna_ref[...]) + _dot(iso, wnb_ref[...])
    r_ref[...] = _dot(m, wra_ref[...]) + _dot(iso, wrb_ref[...]) + b_ref[...]


def _rows_spec(w):
    return pl.BlockSpec((2, RB, w), lambda i: (0, i, 0))


def _full_spec(shape):
    nd = len(shape)
    return pl.BlockSpec(shape, lambda i: (0,) * nd)


def _entry(x, wn, wr, b):
    return pl.pallas_call(
        _entry_body,
        grid=(N // RB,),
        in_specs=[
            pl.BlockSpec((RB, x.shape[1]), lambda i: (i, 0)),
            _full_spec(wn.shape), _full_spec(wr.shape), _full_spec(b.shape),
        ],
        out_specs=[pl.BlockSpec((RB, H), lambda i: (i, 0))] * 2,
        out_shape=[jax.ShapeDtypeStruct((N, H), f32)] * 2,
    )(x, wn, wr, b)


def _mid(p, r, wn, wr, b):
    return pl.pallas_call(
        _mid_body,
        grid=(N // RB,),
        in_specs=[
            _rows_spec(H),
            pl.BlockSpec((RB, H), lambda i: (i, 0)),
            _full_spec(wn.shape), _full_spec(wr.shape), _full_spec(b.shape),
        ],
        out_specs=[pl.BlockSpec((RB, H), lambda i: (i, 0))] * 2,
        out_shape=[jax.ShapeDtypeStruct((N, H), f32)] * 2,
    )(p, r, wn, wr, b)


def _aug(p, r):
    return pl.pallas_call(
        _aug_body,
        grid=(N // RB,),
        in_specs=[_rows_spec(H), pl.BlockSpec((RB, H), lambda i: (i, 0))],
        out_specs=pl.BlockSpec((RB, WAUG), lambda i: (i, 0)),
        out_shape=jax.ShapeDtypeStruct((N, WAUG), f32),
    )(p, r)


def _lvl_entry(p, iso, wna, wnb, wra, wrb, b):
    ni = iso.shape[1]
    return pl.pallas_call(
        _lvl_entry_body,
        grid=(N // RB,),
        in_specs=[
            _rows_spec(WAUG),
            pl.BlockSpec((RB, ni), lambda i: (i, 0)),
            _full_spec(wna.shape), _full_spec(wnb.shape),
            _full_spec(wra.shape), _full_spec(wrb.shape), _full_spec(b.shape),
        ],
        out_specs=[pl.BlockSpec((RB, H), lambda i: (i, 0))] * 2,
        out_shape=[jax.ShapeDtypeStruct((N, H), f32)] * 2,
    )(p, iso, wna, wnb, wra, wrb, b)


def _head_body(x1_ref, x2_ref, x3_ref, f1a_ref, f1b_ref, f1c_ref, b1_ref,
               w2_ref, b2_ref, w3_ref, b3_ref, o_ref):
    a = x1_ref[...]
    x1 = (a[0] + a[1])[:, :H]
    a = x2_ref[...]
    s = a[0] + a[1]
    x2 = s[:, :H] / jnp.maximum(s[:, H:H + 1], 1.0)
    a = x3_ref[...]
    s = a[0] + a[1]
    x3 = s[:, :H] / jnp.maximum(s[:, H:H + 1], 1.0)
    z = _elu(_dot(x1, f1a_ref[...]) + _dot(x2, f1b_ref[...])
             + _dot(x3, f1c_ref[...]) + b1_ref[...])
    z = _elu(_dot(z, w2_ref[...]) + b2_ref[...])
    z = _dot(z, w3_ref[...]) + b3_ref[...]
    m = jnp.max(z, axis=1, keepdims=True)
    lse = jnp.log(jnp.sum(jnp.exp(z - m), axis=1, keepdims=True)) + m
    o_ref[...] = z - lse


def _head(x1s, x2s, x3s, f1a, f1b, f1c, b1, w2, b2, w3, b3):
    args = (x1s, x2s, x3s, f1a, f1b, f1c, b1, w2, b2, w3, b3)

    def spec(shape):
        return pl.BlockSpec(shape, functools.partial(lambda n: (0,) * n, len(shape)))

    return pl.pallas_call(
        _head_body,
        in_specs=[spec(a.shape) for a in args],
        out_specs=pl.BlockSpec((64, 10), lambda: (0, 0)),
        out_shape=jax.ShapeDtypeStruct((64, 10), f32),
    )(*args)


def kernel(x, edge_index, batch, assignment_index_2, iso_type_2, edge_index_2,
           batch_2, assignment_index_3, iso_type_3, edge_index_3, batch_3,
           W1_root, W1_rel, b1, W2_root, W2_rel, b2, W3_root, W3_rel, b3,
           W4_root, W4_rel, b4, W5_root, W5_rel, b5, W6_root, W6_rel, b6,
           W7_root, W7_rel, b7, fc1_W, fc1_b, fc2_W, fc2_b, fc3_W, fc3_b):
    iota = jnp.arange(N, dtype=i32)
    src1, dst1 = edge_index[0], edge_index[1]

    y, r = _entry(x, W1_rel, W1_root, b1.reshape(1, H))
    p = _seg_sum(y, src1, dst1, N)
    y, r = _mid(p, r, W2_rel, W2_root, b2.reshape(1, H))
    p = _seg_sum(y, src1, dst1, N)
    y, r = _mid(p, r, W3_rel, W3_root, b3.reshape(1, H))
    p = _seg_sum(y, src1, dst1, N)
    h_aug = _aug(p, r)

    src_all = jnp.concatenate([iota, assignment_index_2[0], assignment_index_3[0]])
    dst_all = jnp.concatenate([batch, assignment_index_2[1] + 64,
                               assignment_index_3[1] + 10064])
    mega = _seg_sum(h_aug, src_all, dst_all, 20064)
    x1s = mega[:, :64]
    pool2 = mega[:, 64:10064]
    pool3 = mega[:, 10064:20064]

    def level(pool, iso, wroot_e, wrel_e, b_e, wroot_m, wrel_m, b_m, ei, bat):
        y, r = _lvl_entry(pool, iso, wrel_e[:H], wrel_e[H:],
                          wroot_e[:H], wroot_e[H:], b_e.reshape(1, H))
        p = _seg_sum(y, ei[0], ei[1], N)
        y, r = _mid(p, r, wrel_m, wroot_m, b_m.reshape(1, H))
        p = _seg_sum(y, ei[0], ei[1], N)
        haug = _aug(p, r)
        return _seg_sum(haug, iota, bat, 64)[:, :64], p

    x2s, p_l2 = level(pool2, iso_type_2, W4_root, W4_rel, b4, W5_root, W5_rel,
                      b5, edge_index_2, batch_2)
    pool3, _ = lax.optimization_barrier((pool3, p_l2))
    x3s, _ = level(pool3, iso_type_3, W6_root, W6_rel, b6, W7_root, W7_rel, b7,
                   edge_index_3, batch_3)

    return _head(x1s, x2s, x3s,
                 fc1_W[:H], fc1_W[H:2 * H], fc1_W[2 * H:],
                 fc1_b.reshape(1, H), fc2_W, fc2_b.reshape(1, 32),
                 fc3_W, fc3_b.reshape(1, 10))
